# baseline (speedup 1.0000x reference)
import functools

import jax
import jax.numpy as jnp
from jax.experimental import pallas as pl
from jax.experimental.pallas import tpu as pltpu

_PERCENT = 0.33


def _score_body(x_ref, ohc_ref, w_ref, b_ref, y_ref, score_ref):
    C, HW = x_ref.shape[1], x_ref.shape[2]
    xb = x_ref[0]
    ohc = ohc_ref[0]
    w = w_ref[...]
    wl_row = jnp.sum(w * ohc, axis=0, keepdims=True)
    xt = jnp.swapaxes(xb, 0, 1)
    pooled = jnp.sum(xt, axis=0, keepdims=True) * (1.0 / HW)
    yk = jnp.sum(w * pooled, axis=1, keepdims=True)
    y_ref[0] = jnp.swapaxes(yk, 0, 1) + b_ref[...]
    s = xt * wl_row
    mx = jnp.max(s, axis=1, keepdims=True)
    mn = jnp.min(s, axis=1, keepdims=True)
    den = mx - mn
    r = 1.0 / den
    r = r * (2.0 - den * r)
    r = r * (2.0 - den * r)
    sn = (s - mn) * r
    acc = sn[0:8]
    for j in range(1, HW // 8):
        acc = acc + sn[8 * j:8 * j + 8]
    a4 = acc + jnp.concatenate([acc[4:8], acc[0:4]], axis=0)
    a2 = a4 + jnp.concatenate([a4[2:8], a4[0:2]], axis=0)
    a1 = a2 + jnp.concatenate([a2[1:8], a2[0:1]], axis=0)
    score_ref[0] = a1[0:1]


def _mask_body(score_ref, mask_ref, *, drop):
    B, C = score_ref.shape
    s = score_ref[...]
    s = jnp.where(s == 0.0, 0.0, s)
    i = jax.lax.bitcast_convert_type(s, jnp.int32)
    key = i ^ ((i >> 31) & jnp.int32(0x7FFFFFFF))

    def body(_, carry):
        lo, hi = carry
        mid = (lo >> 1) + (hi >> 1) + ((lo | hi) & 1)
        cnt = jnp.sum((key >= mid).astype(jnp.int32), axis=1, keepdims=True)
        ge = cnt >= drop
        return jnp.where(ge, mid, lo), jnp.where(ge, hi, mid - 1)

    lo0 = jnp.full((B, 1), jnp.int32(-2147483648))
    hi0 = jnp.full((B, 1), jnp.int32(2147483647))
    tau, _ = jax.lax.fori_loop(0, 32, body, (lo0, hi0))

    gt = key > tau
    eq = key == tau
    n_gt = jnp.sum(gt.astype(jnp.int32), axis=1, keepdims=True)
    ci = jax.lax.broadcasted_iota(jnp.int32, (C, C), 0)
    cj = jax.lax.broadcasted_iota(jnp.int32, (C, C), 1)
    tri = (ci <= cj).astype(jnp.float32)
    cum = jnp.dot(eq.astype(jnp.float32), tri,
                  preferred_element_type=jnp.float32)
    quota = (jnp.int32(drop) - n_gt).astype(jnp.float32)
    dropped = jnp.logical_or(gt, jnp.logical_and(eq, cum <= quota))
    mask_ref[...] = 1.0 - dropped.astype(jnp.float32)


def kernel(x, labels, W, b):
    B, C, H, Wd = x.shape
    K = W.shape[0]
    HW = H * Wd
    drop = int(C * _PERCENT)
    x3 = x.reshape(B, C, HW)
    oh = (labels.astype(jnp.int32)[:, None]
          == jnp.arange(K, dtype=jnp.int32)[None, :]).astype(jnp.float32)
    ohc = oh.reshape(B, K, 1)
    b2 = b.reshape(1, K).astype(jnp.float32)
    y, score = pl.pallas_call(
        _score_body,
        grid=(B,),
        in_specs=[
            pl.BlockSpec((1, C, HW), lambda i: (i, 0, 0)),
            pl.BlockSpec((1, K, 1), lambda i: (i, 0, 0)),
            pl.BlockSpec((K, C), lambda i: (0, 0)),
            pl.BlockSpec((1, K), lambda i: (0, 0)),
        ],
        out_specs=(
            pl.BlockSpec((1, 1, K), lambda i: (i, 0, 0)),
            pl.BlockSpec((1, 1, C), lambda i: (i, 0, 0)),
        ),
        out_shape=(
            jax.ShapeDtypeStruct((B, 1, K), jnp.float32),
            jax.ShapeDtypeStruct((B, 1, C), jnp.float32),
        ),
        compiler_params=pltpu.CompilerParams(
            dimension_semantics=("parallel",),
        ),
    )(x3, ohc, W, b2)
    mask = pl.pallas_call(
        functools.partial(_mask_body, drop=drop),
        out_shape=jax.ShapeDtypeStruct((B, C), jnp.float32),
    )(score.reshape(B, C))
    return (y.reshape(B, K), mask.reshape(B, C, 1, 1))
